# baseline (speedup 1.0000x reference)
import functools
import math

import jax
import jax.numpy as jnp
from jax import lax
from jax.experimental import pallas as pl
from jax.experimental.pallas import tpu as pltpu
from jax.experimental.pallas import tpu_sc as plsc

N = 10000
D = 256
H = 8
DH = 32
F = 512
E = 160000
L = 3

NC = 2
NS = 16
HALF = N // NC
RPAD = 5008
NZCH = RPAD // 16
NW = NC * NS
SLICE = 5024
EPAD = SLICE * NW
CH = 32
SCAP = 5056
ISQ = 1.0 / math.sqrt(DH)
DUMP = HALF

_f32 = jnp.float32
_i32 = jnp.int32


def _part_body(src_hbm, dst_hbm, srcp_hbm, dstp_hbm, cnt_hbm,
               sin, din, os0, od0, os1, od1, cb0, cb1):
    c = lax.axis_index("c")
    sid = lax.axis_index("s")
    w = c * NS + sid
    iota = lax.iota(_i32, 16)

    def _pre(i, _):
        sl = pl.ds(i * 16, 16)
        os0[sl] = jnp.zeros((16,), _i32)
        os1[sl] = jnp.zeros((16,), _i32)
        od0[sl] = jnp.full((16,), N, _i32)
        od1[sl] = jnp.full((16,), N, _i32)
        return 0
    lax.fori_loop(0, SCAP // 16, _pre, 0)

    pltpu.sync_copy(src_hbm.at[pl.ds(w * SLICE, SLICE)], sin)
    pltpu.sync_copy(dst_hbm.at[pl.ds(w * SLICE, SLICE)], din)

    def _grp(i, carry):
        o0, o1 = carry
        sv = sin[pl.ds(i * 16, 16)]
        dv = din[pl.ds(i * 16, 16)]
        m0 = dv < HALF
        plsc.store_compressed(os0.at[pl.ds(o0, 16)], sv, mask=m0)
        plsc.store_compressed(od0.at[pl.ds(o0, 16)], dv, mask=m0)
        plsc.store_compressed(os1.at[pl.ds(o1, 16)], sv, mask=~m0)
        plsc.store_compressed(od1.at[pl.ds(o1, 16)], dv, mask=~m0)
        n0 = jnp.max(plsc.all_reduce_population_count(m0))
        return (o0 + n0, o1 + (16 - n0))

    o0, o1 = lax.fori_loop(0, SLICE // 16, _grp, (0, 0))
    cb0[...] = jnp.full((16,), 1, _i32) * o0
    cb1[...] = jnp.full((16,), 1, _i32) * o1
    pltpu.sync_copy(os0, srcp_hbm.at[0, w])
    pltpu.sync_copy(od0, dstp_hbm.at[0, w])
    pltpu.sync_copy(os1, srcp_hbm.at[1, w])
    pltpu.sync_copy(od1, dstp_hbm.at[1, w])
    pltpu.sync_copy(cb0, cnt_hbm.at[0, w])
    pltpu.sync_copy(cb1, cnt_hbm.at[1, w])


@jax.jit
def _sc_part(src, dst):
    mesh = plsc.VectorSubcoreMesh(core_axis_name="c", subcore_axis_name="s",
                                  num_cores=NC, num_subcores=NS)
    f = pl.kernel(
        _part_body,
        out_type=[
            jax.ShapeDtypeStruct((NC, NW, SCAP), _i32),
            jax.ShapeDtypeStruct((NC, NW, SCAP), _i32),
            jax.ShapeDtypeStruct((NC, NW, 16), _i32),
        ],
        mesh=mesh,
        compiler_params=pltpu.CompilerParams(use_tc_tiling_on_sc=False,
                                             needs_layout_passes=False),
        scratch_types=[
            pltpu.VMEM((SLICE,), _i32),
            pltpu.VMEM((SLICE,), _i32),
            pltpu.VMEM((SCAP,), _i32),
            pltpu.VMEM((SCAP,), _i32),
            pltpu.VMEM((SCAP,), _i32),
            pltpu.VMEM((SCAP,), _i32),
            pltpu.VMEM((16,), _i32),
            pltpu.VMEM((16,), _i32),
        ],
    )
    return f(src, dst)


def _edge_body(q_hbm, k_hbm, v_hbm, srcp_hbm, dstp_hbm, cnt_hbm,
               zacc_hbm, zden_hbm, acc_hbm, den_hbm,
               qbufs, kbufs, vbuf, exrow, sidxs, didxs, dloc, cntb,
               semg, semv, acc_sp, den_sp):
    c = lax.axis_index("c")
    s = lax.axis_index("s")
    iota = lax.iota(_i32, 16)
    zv = jnp.zeros((16,), _f32)
    nbase = c * HALF

    for t in range(-(-NZCH // NS)):
        ch = t * NS + s
        @pl.when(ch < NZCH)
        def _():
            pltpu.sync_copy(zacc_hbm, acc_sp.at[pl.ds(ch * 16, 16)])
            pltpu.sync_copy(zden_hbm, den_sp.at[pl.ds(ch * 16, 16)])

    def _zex(i, _):
        exrow[i, :] = zv
        return 0
    lax.fori_loop(0, CH, _zex, 0)

    pltpu.sync_copy(cnt_hbm.at[c], cntb)
    plsc.subcore_barrier()

    def _alpha(qbuf, kbuf):
        def _mh(m, _):
            g = m // H
            h = m - g * H
            rowv = g * 16 + iota
            basec = jnp.full((16,), h * DH, _i32)

            def _dd(dd, acc):
                colv = basec + ((dd + iota) & (DH - 1))
                qd = plsc.load_gather(qbuf, [rowv, colv])
                kd = plsc.load_gather(kbuf, [rowv, colv])
                return acc + qd * kd

            acc = lax.fori_loop(0, DH, _dd, zv, unroll=DH)
            ex = jnp.exp(acc * ISQ)
            plsc.store_scatter(exrow, [rowv, jnp.full((16,), h, _i32)], ex)
            return 0

        lax.fori_loop(0, (CH // 16) * H, _mh, 0, unroll=4)

    def _msg():
        def _mh2(m, _):
            g = m // H
            h = m - g * H
            rowv = g * 16 + iota
            basec = jnp.full((16,), h * DH, _i32)
            exv = plsc.load_gather(exrow, [rowv, jnp.full((16,), h, _i32)])

            def _dd(dd, _):
                colv = basec + ((dd + iota) & (DH - 1))
                vd = plsc.load_gather(vbuf, [rowv, colv])
                plsc.store_scatter(vbuf, [rowv, colv], vd * exv)
                return 0

            lax.fori_loop(0, DH, _dd, 0, unroll=DH)
            return 0

        lax.fori_loop(0, (CH // 16) * H, _mh2, 0, unroll=4)

    def _run_row(row):
        cl = cntb[row, :]
        cnt = jnp.max(cl)
        nch = jnp.maximum((cnt + (CH - 1)) // CH, 1)
        npair = (nch + 1) // 2
        nst = 2 * npair

        def _fetch_idx(i, y):
            pltpu.sync_copy(srcp_hbm.at[c, row, pl.ds(i * CH, CH)], sidxs[y])
            pltpu.sync_copy(dstp_hbm.at[c, row, pl.ds(i * CH, CH)], didxs[y])

        def _step(i, x, pc):
            y = 1 - x
            pltpu.make_async_copy(q_hbm.at[didxs[x]], qbufs[x], semg[x]).wait()
            pltpu.make_async_copy(k_hbm.at[sidxs[x]], kbufs[x], semg[x]).wait()
            _alpha(qbufs[x], kbufs[x])

            @pl.when(pc)
            def _():
                _fetch_idx(i + 1, y)
                pltpu.async_copy(q_hbm.at[didxs[y]], qbufs[y], semg[y])
                pltpu.async_copy(k_hbm.at[sidxs[y]], kbufs[y], semg[y])

            pltpu.make_async_copy(v_hbm.at[sidxs[x]], vbuf, semv).wait()
            _msg()
            for g in range(CH // 16):
                dv = didxs[x][pl.ds(g * 16, 16)] - nbase
                inb = (dv >= 0) & (dv < HALF)
                dloc[pl.ds(g * 16, 16)] = jnp.where(inb, dv, DUMP)
            pltpu.sync_copy(vbuf, acc_sp.at[dloc], add=True)
            pltpu.sync_copy(exrow, den_sp.at[dloc], add=True)

            @pl.when(pc)
            def _():
                pltpu.async_copy(v_hbm.at[sidxs[y]], vbuf, semv)

        _fetch_idx(0, 0)
        pltpu.async_copy(q_hbm.at[didxs[0]], qbufs[0], semg[0])
        pltpu.async_copy(k_hbm.at[sidxs[0]], kbufs[0], semg[0])
        pltpu.async_copy(v_hbm.at[sidxs[0]], vbuf, semv)

        def _pair(j, _):
            i0 = 2 * j
            _step(i0, 0, i0 + 1 < nst)
            _step(i0 + 1, 1, j + 1 < npair)
            return 0

        lax.fori_loop(0, npair, _pair, 0)

    _run_row(2 * s)
    _run_row(2 * s + 1)

    plsc.subcore_barrier()

    for t in range(-(-NZCH // NS)):
        ch = t * NS + s
        @pl.when(ch < NZCH)
        def _():
            pltpu.sync_copy(acc_sp.at[pl.ds(ch * 16, 16)],
                            acc_hbm.at[c, pl.ds(ch * 16, 16)])
            pltpu.sync_copy(den_sp.at[pl.ds(ch * 16, 16)],
                            den_hbm.at[c, pl.ds(ch * 16, 16)])


@jax.jit
def _sc_edge(q, k, v, srcp, dstp, cnt):
    mesh = plsc.VectorSubcoreMesh(core_axis_name="c", subcore_axis_name="s",
                                  num_cores=NC, num_subcores=NS)
    f = pl.kernel(
        _edge_body,
        out_type=[
            jax.ShapeDtypeStruct((NC, RPAD, D), _f32),
            jax.ShapeDtypeStruct((NC, RPAD, 16), _f32),
        ],
        mesh=mesh,
        compiler_params=pltpu.CompilerParams(use_tc_tiling_on_sc=False,
                                             needs_layout_passes=False),
        scratch_types=[
            [pltpu.VMEM((CH, D), _f32)] * 2,
            [pltpu.VMEM((CH, D), _f32)] * 2,
            pltpu.VMEM((CH, D), _f32),
            pltpu.VMEM((CH, 16), _f32),
            [pltpu.VMEM((CH,), _i32)] * 2,
            [pltpu.VMEM((CH,), _i32)] * 2,
            pltpu.VMEM((CH,), _i32),
            pltpu.VMEM((NW, 16), _i32),
            [pltpu.SemaphoreType.DMA] * 2,
            pltpu.SemaphoreType.DMA,
            pltpu.VMEM_SHARED((RPAD, D), _f32),
            pltpu.VMEM_SHARED((RPAD, 16), _f32),
        ],
    )
    zacc = jnp.zeros((16, D), _f32)
    zden = jnp.zeros((16, 16), _f32)
    return f(q, k, v, srcp, dstp, cnt, zacc, zden)


RB = 400
NB = N // RB


def _bn_affine(ps, pss, g, b):
    mean = jnp.sum(ps, axis=(0, 1)).reshape(1, D) / N
    ex2 = jnp.sum(pss, axis=(0, 1)).reshape(1, D) / N
    var = ex2 - mean * mean
    scale = g / jnp.sqrt(var + 1e-5)
    return scale, b - mean * scale


def _qkvs_body(z_ref, ps_ref, pss_ref, g_ref, b_ref, wq_ref, bq_ref,
               wk_ref, bk_ref, wv_ref, bv_ref, ws_ref, bs_ref,
               q_ref, k_ref, v_ref, xs_ref):
    sc, sh = _bn_affine(ps_ref[...], pss_ref[...], g_ref[...], b_ref[...])
    xn = z_ref[...] * sc + sh
    q_ref[...] = jnp.dot(xn, wq_ref[...], preferred_element_type=_f32) + bq_ref[...]
    k_ref[...] = jnp.dot(xn, wk_ref[...], preferred_element_type=_f32) + bk_ref[...]
    v_ref[...] = jnp.dot(xn, wv_ref[...], preferred_element_type=_f32) + bv_ref[...]
    xs_ref[...] = jnp.dot(xn, ws_ref[...], preferred_element_type=_f32) + bs_ref[...]


def _qkvs(z, ps, pss, g, b, wq, bq, wk, bk, wv, bv, ws, bs):
    full = pl.BlockSpec((D, D), lambda i: (0, 0))
    row1 = pl.BlockSpec((1, D), lambda i: (0, 0))
    blk = pl.BlockSpec((RB, D), lambda i: (i, 0))
    nps = ps.shape[0]
    psf = pl.BlockSpec((nps, 1, D), lambda i: (0, 0, 0))
    return pl.pallas_call(
        _qkvs_body,
        grid=(NB,),
        in_specs=[blk, psf, psf, row1, row1, full, row1, full, row1, full,
                  row1, full, row1],
        out_specs=[blk, blk, blk, blk],
        out_shape=[jax.ShapeDtypeStruct((N, D), _f32)] * 4,
    )(z, ps, pss, g, b, wq, bq, wk, bk, wv, bv, ws, bs)


def _comb_body(acc_ref, den_ref, xs_ref, y_ref, ps_ref, pss_ref):
    a = acc_ref[0].reshape(RBC, H, DH)
    d = den_ref[0][:, :H]
    safe = jnp.where(d > 0.0, d, 1.0)
    msg = jnp.where(d[:, :, None] > 0.0, a / safe[:, :, None], 0.0)
    y = msg.reshape(RBC, D) + xs_ref[...]
    y_ref[...] = y
    ps_ref[0] = jnp.sum(y, axis=0, keepdims=True)
    pss_ref[0] = jnp.sum(y * y, axis=0, keepdims=True)


RBC = 200
NBC = N // RBC


def _combine(acc, den, xs):
    per = HALF // RBC
    return pl.pallas_call(
        _comb_body,
        grid=(NBC,),
        in_specs=[
            pl.BlockSpec((1, RBC, D), lambda b: (b // per, b % per, 0)),
            pl.BlockSpec((1, RBC, 16), lambda b: (b // per, b % per, 0)),
            pl.BlockSpec((RBC, D), lambda b: (b, 0)),
        ],
        out_specs=[
            pl.BlockSpec((RBC, D), lambda b: (b, 0)),
            pl.BlockSpec((1, 1, D), lambda b: (b, 0, 0)),
            pl.BlockSpec((1, 1, D), lambda b: (b, 0, 0)),
        ],
        out_shape=[
            jax.ShapeDtypeStruct((N, D), _f32),
            jax.ShapeDtypeStruct((NBC, 1, D), _f32),
            jax.ShapeDtypeStruct((NBC, 1, D), _f32),
        ],
    )(acc, den, xs)


def _ffn_body(y_ref, psi_ref, pssi_ref, g_ref, b_ref, w1_ref, b1_ref,
              w2_ref, b2_ref, z_ref, ps_ref, pss_ref):
    sc, sh = _bn_affine(psi_ref[...], pssi_ref[...], g_ref[...], b_ref[...])
    xb = y_ref[...] * sc + sh
    h = jnp.dot(xb, w1_ref[...], preferred_element_type=_f32) + b1_ref[...]
    h = jnp.maximum(h, 0.0)
    z = jnp.dot(h, w2_ref[...], preferred_element_type=_f32) + b2_ref[...]
    z_ref[...] = z
    ps_ref[0] = jnp.sum(z, axis=0, keepdims=True)
    pss_ref[0] = jnp.sum(z * z, axis=0, keepdims=True)


def _ffn(y, psi, pssi, g, b, w1, b1, w2, b2):
    return pl.pallas_call(
        _ffn_body,
        grid=(NB,),
        in_specs=[
            pl.BlockSpec((RB, D), lambda i: (i, 0)),
            pl.BlockSpec((NBC, 1, D), lambda i: (0, 0, 0)),
            pl.BlockSpec((NBC, 1, D), lambda i: (0, 0, 0)),
            pl.BlockSpec((1, D), lambda i: (0, 0)),
            pl.BlockSpec((1, D), lambda i: (0, 0)),
            pl.BlockSpec((D, F), lambda i: (0, 0)),
            pl.BlockSpec((1, F), lambda i: (0, 0)),
            pl.BlockSpec((F, D), lambda i: (0, 0)),
            pl.BlockSpec((1, D), lambda i: (0, 0)),
        ],
        out_specs=[
            pl.BlockSpec((RB, D), lambda b: (b, 0)),
            pl.BlockSpec((1, 1, D), lambda b: (b, 0, 0)),
            pl.BlockSpec((1, 1, D), lambda b: (b, 0, 0)),
        ],
        out_shape=[
            jax.ShapeDtypeStruct((N, D), _f32),
            jax.ShapeDtypeStruct((NB, 1, D), _f32),
            jax.ShapeDtypeStruct((NB, 1, D), _f32),
        ],
    )(y, psi, pssi, g, b, w1, b1, w2, b2)


def _apply_body(z_ref, ps_ref, pss_ref, g_ref, b_ref, o_ref):
    sc, sh = _bn_affine(ps_ref[...], pss_ref[...], g_ref[...], b_ref[...])
    o_ref[...] = z_ref[...] * sc + sh


def _apply(z, ps, pss, g, b):
    return pl.pallas_call(
        _apply_body,
        grid=(NB,),
        in_specs=[
            pl.BlockSpec((RB, D), lambda i: (i, 0)),
            pl.BlockSpec((NB, 1, D), lambda i: (0, 0, 0)),
            pl.BlockSpec((NB, 1, D), lambda i: (0, 0, 0)),
            pl.BlockSpec((1, D), lambda i: (0, 0)),
            pl.BlockSpec((1, D), lambda i: (0, 0)),
        ],
        out_specs=pl.BlockSpec((RB, D), lambda i: (i, 0)),
        out_shape=jax.ShapeDtypeStruct((N, D), _f32),
    )(z, ps, pss, g, b)


def kernel(x, edge_index, Wq, bq, Wk, bk, Wv, bv, Ws, bs, bn1_g, bn1_b,
           W1, b1, W2, b2, bn2_g, bn2_b):
    src = edge_index[0].astype(_i32)
    dst = edge_index[1].astype(_i32)
    pad = EPAD - E
    src = jnp.concatenate([src, jnp.zeros((pad,), _i32)])
    dst = jnp.concatenate([dst, jnp.full((pad,), N, _i32)])
    srcp, dstp, ecnt = _sc_part(src, dst)
    z = x
    ps2 = jnp.zeros((1, 1, D), _f32)
    pss2 = jnp.full((1, 1, D), N * (1.0 - 1e-5), _f32)
    gg = jnp.ones((1, D), _f32)
    bb = jnp.zeros((1, D), _f32)
    for l in range(L):
        q, k, v, xs = _qkvs(z, ps2, pss2, gg, bb,
                            Wq[l], bq[l].reshape(1, D),
                            Wk[l], bk[l].reshape(1, D),
                            Wv[l], bv[l].reshape(1, D),
                            Ws[l], bs[l].reshape(1, D))
        acc, den = _sc_edge(q, k, v, srcp, dstp, ecnt)
        y, ps, pss = _combine(acc, den, xs)
        z, ps2, pss2 = _ffn(y, ps, pss, bn1_g[l].reshape(1, D),
                            bn1_b[l].reshape(1, D), W1[l],
                            b1[l].reshape(1, F), W2[l], b2[l].reshape(1, D))
        gg = bn2_g[l].reshape(1, D)
        bb = bn2_b[l].reshape(1, D)
    return _apply(z, ps2, pss2, gg, bb)


# speedup vs baseline: 1.0587x; 1.0587x over previous
import functools
import math

import jax
import jax.numpy as jnp
from jax import lax
from jax.experimental import pallas as pl
from jax.experimental.pallas import tpu as pltpu
from jax.experimental.pallas import tpu_sc as plsc

N = 10000
D = 256
H = 8
DH = 32
F = 512
E = 160000
L = 3

NC = 2
NS = 16
HALF = N // NC
RPAD = 5008
NZCH = RPAD // 16
NW = NC * NS
SLICE = 5024
EPAD = SLICE * NW
CH = 32
SCAP = 5056
ISQ = 1.0 / math.sqrt(DH)
DUMP = HALF

_f32 = jnp.float32
_i32 = jnp.int32


def _part_body(src_hbm, dst_hbm, srcp_hbm, dstp_hbm, cnt_hbm,
               sin, din, os0, od0, os1, od1, cb0, cb1):
    c = lax.axis_index("c")
    sid = lax.axis_index("s")
    w = c * NS + sid
    iota = lax.iota(_i32, 16)

    def _pre(i, _):
        sl = pl.ds(i * 16, 16)
        os0[sl] = jnp.zeros((16,), _i32)
        os1[sl] = jnp.zeros((16,), _i32)
        od0[sl] = jnp.full((16,), N, _i32)
        od1[sl] = jnp.full((16,), N, _i32)
        return 0
    lax.fori_loop(0, SCAP // 16, _pre, 0)

    pltpu.sync_copy(src_hbm.at[pl.ds(w * SLICE, SLICE)], sin)
    pltpu.sync_copy(dst_hbm.at[pl.ds(w * SLICE, SLICE)], din)

    def _grp(i, carry):
        o0, o1 = carry
        sv = sin[pl.ds(i * 16, 16)]
        dv = din[pl.ds(i * 16, 16)]
        m0 = dv < HALF
        plsc.store_compressed(os0.at[pl.ds(o0, 16)], sv, mask=m0)
        plsc.store_compressed(od0.at[pl.ds(o0, 16)], dv, mask=m0)
        plsc.store_compressed(os1.at[pl.ds(o1, 16)], sv, mask=~m0)
        plsc.store_compressed(od1.at[pl.ds(o1, 16)], dv, mask=~m0)
        n0 = jnp.max(plsc.all_reduce_population_count(m0))
        return (o0 + n0, o1 + (16 - n0))

    o0, o1 = lax.fori_loop(0, SLICE // 16, _grp, (0, 0))
    cb0[...] = jnp.full((16,), 1, _i32) * o0
    cb1[...] = jnp.full((16,), 1, _i32) * o1
    pltpu.sync_copy(os0, srcp_hbm.at[0, w])
    pltpu.sync_copy(od0, dstp_hbm.at[0, w])
    pltpu.sync_copy(os1, srcp_hbm.at[1, w])
    pltpu.sync_copy(od1, dstp_hbm.at[1, w])
    pltpu.sync_copy(cb0, cnt_hbm.at[0, w])
    pltpu.sync_copy(cb1, cnt_hbm.at[1, w])


@jax.jit
def _sc_part(src, dst):
    mesh = plsc.VectorSubcoreMesh(core_axis_name="c", subcore_axis_name="s",
                                  num_cores=NC, num_subcores=NS)
    f = pl.kernel(
        _part_body,
        out_type=[
            jax.ShapeDtypeStruct((NC, NW, SCAP), _i32),
            jax.ShapeDtypeStruct((NC, NW, SCAP), _i32),
            jax.ShapeDtypeStruct((NC, NW, 16), _i32),
        ],
        mesh=mesh,
        compiler_params=pltpu.CompilerParams(use_tc_tiling_on_sc=False,
                                             needs_layout_passes=False),
        scratch_types=[
            pltpu.VMEM((SLICE,), _i32),
            pltpu.VMEM((SLICE,), _i32),
            pltpu.VMEM((SCAP,), _i32),
            pltpu.VMEM((SCAP,), _i32),
            pltpu.VMEM((SCAP,), _i32),
            pltpu.VMEM((SCAP,), _i32),
            pltpu.VMEM((16,), _i32),
            pltpu.VMEM((16,), _i32),
        ],
    )
    return f(src, dst)


def _edge_body(q_hbm, k_hbm, v_hbm, srcp_hbm, dstp_hbm, cnt_hbm,
               zacc_hbm, zden_hbm, acc_hbm, den_hbm,
               qbufs, kbufs, vbuf, exrow, sidxs, didxs, dloc, cntb,
               semg, semv, acc_sp, den_sp):
    c = lax.axis_index("c")
    s = lax.axis_index("s")
    iota = lax.iota(_i32, 16)
    zv = jnp.zeros((16,), _f32)
    nbase = c * HALF

    for t in range(-(-NZCH // NS)):
        ch = t * NS + s
        @pl.when(ch < NZCH)
        def _():
            pltpu.sync_copy(zacc_hbm, acc_sp.at[pl.ds(ch * 16, 16)])
            pltpu.sync_copy(zden_hbm, den_sp.at[pl.ds(ch * 16, 16)])

    def _zex(i, _):
        exrow[i, :] = zv
        return 0
    lax.fori_loop(0, CH, _zex, 0)

    pltpu.sync_copy(cnt_hbm.at[c], cntb)
    plsc.subcore_barrier()

    def _alpha(qbuf, kbuf):
        def _mh(m, _):
            g = m // H
            h = m - g * H
            rowv = g * 16 + iota
            basec = jnp.full((16,), h * DH, _i32)

            def _dd(dd, acc):
                colv = basec + ((dd + iota) & (DH - 1))
                qd = plsc.load_gather(qbuf, [rowv, colv])
                kd = plsc.load_gather(kbuf, [rowv, colv])
                return acc + qd * kd

            acc = lax.fori_loop(0, DH, _dd, zv, unroll=DH)
            ex = jnp.exp(acc * ISQ)
            plsc.store_scatter(exrow, [rowv, jnp.full((16,), h, _i32)], ex)
            return 0

        lax.fori_loop(0, (CH // 16) * H, _mh, 0, unroll=2)

    def _msg():
        def _mh2(m, _):
            g = m // H
            h = m - g * H
            rowv = g * 16 + iota
            basec = jnp.full((16,), h * DH, _i32)
            exv = plsc.load_gather(exrow, [rowv, jnp.full((16,), h, _i32)])

            def _dd(dd, _):
                colv = basec + ((dd + iota) & (DH - 1))
                vd = plsc.load_gather(vbuf, [rowv, colv])
                plsc.store_scatter(vbuf, [rowv, colv], vd * exv)
                return 0

            lax.fori_loop(0, DH, _dd, 0, unroll=DH)
            return 0

        lax.fori_loop(0, (CH // 16) * H, _mh2, 0, unroll=2)

    def _run_row(row):
        cl = cntb[row, :]
        cnt = jnp.max(cl)
        nch = jnp.maximum((cnt + (CH - 1)) // CH, 1)
        npair = (nch + 1) // 2
        nst = 2 * npair

        def _fetch_idx(i, y):
            pltpu.sync_copy(srcp_hbm.at[c, row, pl.ds(i * CH, CH)], sidxs[y])
            pltpu.sync_copy(dstp_hbm.at[c, row, pl.ds(i * CH, CH)], didxs[y])

        def _step(i, x, pc):
            y = 1 - x
            pltpu.make_async_copy(q_hbm.at[didxs[x]], qbufs[x], semg[x]).wait()
            pltpu.make_async_copy(k_hbm.at[sidxs[x]], kbufs[x], semg[x]).wait()
            _alpha(qbufs[x], kbufs[x])

            @pl.when(pc)
            def _():
                _fetch_idx(i + 1, y)
                pltpu.async_copy(q_hbm.at[didxs[y]], qbufs[y], semg[y])
                pltpu.async_copy(k_hbm.at[sidxs[y]], kbufs[y], semg[y])

            pltpu.make_async_copy(v_hbm.at[sidxs[x]], vbuf, semv).wait()
            _msg()
            for g in range(CH // 16):
                dv = didxs[x][pl.ds(g * 16, 16)] - nbase
                inb = (dv >= 0) & (dv < HALF)
                dloc[pl.ds(g * 16, 16)] = jnp.where(inb, dv, DUMP)
            pltpu.sync_copy(vbuf, acc_sp.at[dloc], add=True)
            pltpu.sync_copy(exrow, den_sp.at[dloc], add=True)

            @pl.when(pc)
            def _():
                pltpu.async_copy(v_hbm.at[sidxs[y]], vbuf, semv)

        _fetch_idx(0, 0)
        pltpu.async_copy(q_hbm.at[didxs[0]], qbufs[0], semg[0])
        pltpu.async_copy(k_hbm.at[sidxs[0]], kbufs[0], semg[0])
        pltpu.async_copy(v_hbm.at[sidxs[0]], vbuf, semv)

        def _pair(j, _):
            i0 = 2 * j
            _step(i0, 0, i0 + 1 < nst)
            _step(i0 + 1, 1, j + 1 < npair)
            return 0

        lax.fori_loop(0, npair, _pair, 0)

    _run_row(2 * s)
    _run_row(2 * s + 1)

    plsc.subcore_barrier()

    for t in range(-(-NZCH // NS)):
        ch = t * NS + s
        @pl.when(ch < NZCH)
        def _():
            pltpu.sync_copy(acc_sp.at[pl.ds(ch * 16, 16)],
                            acc_hbm.at[c, pl.ds(ch * 16, 16)])
            pltpu.sync_copy(den_sp.at[pl.ds(ch * 16, 16)],
                            den_hbm.at[c, pl.ds(ch * 16, 16)])


@jax.jit
def _sc_edge(q, k, v, srcp, dstp, cnt):
    mesh = plsc.VectorSubcoreMesh(core_axis_name="c", subcore_axis_name="s",
                                  num_cores=NC, num_subcores=NS)
    f = pl.kernel(
        _edge_body,
        out_type=[
            jax.ShapeDtypeStruct((NC, RPAD, D), _f32),
            jax.ShapeDtypeStruct((NC, RPAD, 16), _f32),
        ],
        mesh=mesh,
        compiler_params=pltpu.CompilerParams(use_tc_tiling_on_sc=False,
                                             needs_layout_passes=False),
        scratch_types=[
            [pltpu.VMEM((CH, D), _f32)] * 2,
            [pltpu.VMEM((CH, D), _f32)] * 2,
            pltpu.VMEM((CH, D), _f32),
            pltpu.VMEM((CH, 16), _f32),
            [pltpu.VMEM((CH,), _i32)] * 2,
            [pltpu.VMEM((CH,), _i32)] * 2,
            pltpu.VMEM((CH,), _i32),
            pltpu.VMEM((NW, 16), _i32),
            [pltpu.SemaphoreType.DMA] * 2,
            pltpu.SemaphoreType.DMA,
            pltpu.VMEM_SHARED((RPAD, D), _f32),
            pltpu.VMEM_SHARED((RPAD, 16), _f32),
        ],
    )
    zacc = jnp.zeros((16, D), _f32)
    zden = jnp.zeros((16, 16), _f32)
    return f(q, k, v, srcp, dstp, cnt, zacc, zden)


RB = 400
NB = N // RB


def _bn_affine(ps, pss, g, b):
    mean = jnp.sum(ps, axis=(0, 1)).reshape(1, D) / N
    ex2 = jnp.sum(pss, axis=(0, 1)).reshape(1, D) / N
    var = ex2 - mean * mean
    scale = g / jnp.sqrt(var + 1e-5)
    return scale, b - mean * scale


def _qkvs_body(z_ref, ps_ref, pss_ref, g_ref, b_ref, wq_ref, bq_ref,
               wk_ref, bk_ref, wv_ref, bv_ref, ws_ref, bs_ref,
               q_ref, k_ref, v_ref, xs_ref):
    sc, sh = _bn_affine(ps_ref[...], pss_ref[...], g_ref[...], b_ref[...])
    xn = z_ref[...] * sc + sh
    q_ref[...] = jnp.dot(xn, wq_ref[...], preferred_element_type=_f32) + bq_ref[...]
    k_ref[...] = jnp.dot(xn, wk_ref[...], preferred_element_type=_f32) + bk_ref[...]
    v_ref[...] = jnp.dot(xn, wv_ref[...], preferred_element_type=_f32) + bv_ref[...]
    xs_ref[...] = jnp.dot(xn, ws_ref[...], preferred_element_type=_f32) + bs_ref[...]


def _qkvs(z, ps, pss, g, b, wq, bq, wk, bk, wv, bv, ws, bs):
    full = pl.BlockSpec((D, D), lambda i: (0, 0))
    row1 = pl.BlockSpec((1, D), lambda i: (0, 0))
    blk = pl.BlockSpec((RB, D), lambda i: (i, 0))
    nps = ps.shape[0]
    psf = pl.BlockSpec((nps, 1, D), lambda i: (0, 0, 0))
    return pl.pallas_call(
        _qkvs_body,
        grid=(NB,),
        in_specs=[blk, psf, psf, row1, row1, full, row1, full, row1, full,
                  row1, full, row1],
        out_specs=[blk, blk, blk, blk],
        out_shape=[jax.ShapeDtypeStruct((N, D), _f32)] * 4,
    )(z, ps, pss, g, b, wq, bq, wk, bk, wv, bv, ws, bs)


def _comb_body(acc_ref, den_ref, xs_ref, y_ref, ps_ref, pss_ref):
    a = acc_ref[0].reshape(RBC, H, DH)
    d = den_ref[0][:, :H]
    safe = jnp.where(d > 0.0, d, 1.0)
    msg = jnp.where(d[:, :, None] > 0.0, a / safe[:, :, None], 0.0)
    y = msg.reshape(RBC, D) + xs_ref[...]
    y_ref[...] = y
    ps_ref[0] = jnp.sum(y, axis=0, keepdims=True)
    pss_ref[0] = jnp.sum(y * y, axis=0, keepdims=True)


RBC = 200
NBC = N // RBC


def _combine(acc, den, xs):
    per = HALF // RBC
    return pl.pallas_call(
        _comb_body,
        grid=(NBC,),
        in_specs=[
            pl.BlockSpec((1, RBC, D), lambda b: (b // per, b % per, 0)),
            pl.BlockSpec((1, RBC, 16), lambda b: (b // per, b % per, 0)),
            pl.BlockSpec((RBC, D), lambda b: (b, 0)),
        ],
        out_specs=[
            pl.BlockSpec((RBC, D), lambda b: (b, 0)),
            pl.BlockSpec((1, 1, D), lambda b: (b, 0, 0)),
            pl.BlockSpec((1, 1, D), lambda b: (b, 0, 0)),
        ],
        out_shape=[
            jax.ShapeDtypeStruct((N, D), _f32),
            jax.ShapeDtypeStruct((NBC, 1, D), _f32),
            jax.ShapeDtypeStruct((NBC, 1, D), _f32),
        ],
    )(acc, den, xs)


def _ffn_body(y_ref, psi_ref, pssi_ref, g_ref, b_ref, w1_ref, b1_ref,
              w2_ref, b2_ref, z_ref, ps_ref, pss_ref):
    sc, sh = _bn_affine(psi_ref[...], pssi_ref[...], g_ref[...], b_ref[...])
    xb = y_ref[...] * sc + sh
    h = jnp.dot(xb, w1_ref[...], preferred_element_type=_f32) + b1_ref[...]
    h = jnp.maximum(h, 0.0)
    z = jnp.dot(h, w2_ref[...], preferred_element_type=_f32) + b2_ref[...]
    z_ref[...] = z
    ps_ref[0] = jnp.sum(z, axis=0, keepdims=True)
    pss_ref[0] = jnp.sum(z * z, axis=0, keepdims=True)


def _ffn(y, psi, pssi, g, b, w1, b1, w2, b2):
    return pl.pallas_call(
        _ffn_body,
        grid=(NB,),
        in_specs=[
            pl.BlockSpec((RB, D), lambda i: (i, 0)),
            pl.BlockSpec((NBC, 1, D), lambda i: (0, 0, 0)),
            pl.BlockSpec((NBC, 1, D), lambda i: (0, 0, 0)),
            pl.BlockSpec((1, D), lambda i: (0, 0)),
            pl.BlockSpec((1, D), lambda i: (0, 0)),
            pl.BlockSpec((D, F), lambda i: (0, 0)),
            pl.BlockSpec((1, F), lambda i: (0, 0)),
            pl.BlockSpec((F, D), lambda i: (0, 0)),
            pl.BlockSpec((1, D), lambda i: (0, 0)),
        ],
        out_specs=[
            pl.BlockSpec((RB, D), lambda b: (b, 0)),
            pl.BlockSpec((1, 1, D), lambda b: (b, 0, 0)),
            pl.BlockSpec((1, 1, D), lambda b: (b, 0, 0)),
        ],
        out_shape=[
            jax.ShapeDtypeStruct((N, D), _f32),
            jax.ShapeDtypeStruct((NB, 1, D), _f32),
            jax.ShapeDtypeStruct((NB, 1, D), _f32),
        ],
    )(y, psi, pssi, g, b, w1, b1, w2, b2)


def _apply_body(z_ref, ps_ref, pss_ref, g_ref, b_ref, o_ref):
    sc, sh = _bn_affine(ps_ref[...], pss_ref[...], g_ref[...], b_ref[...])
    o_ref[...] = z_ref[...] * sc + sh


def _apply(z, ps, pss, g, b):
    return pl.pallas_call(
        _apply_body,
        grid=(NB,),
        in_specs=[
            pl.BlockSpec((RB, D), lambda i: (i, 0)),
            pl.BlockSpec((NB, 1, D), lambda i: (0, 0, 0)),
            pl.BlockSpec((NB, 1, D), lambda i: (0, 0, 0)),
            pl.BlockSpec((1, D), lambda i: (0, 0)),
            pl.BlockSpec((1, D), lambda i: (0, 0)),
        ],
        out_specs=pl.BlockSpec((RB, D), lambda i: (i, 0)),
        out_shape=jax.ShapeDtypeStruct((N, D), _f32),
    )(z, ps, pss, g, b)


def kernel(x, edge_index, Wq, bq, Wk, bk, Wv, bv, Ws, bs, bn1_g, bn1_b,
           W1, b1, W2, b2, bn2_g, bn2_b):
    src = edge_index[0].astype(_i32)
    dst = edge_index[1].astype(_i32)
    pad = EPAD - E
    src = jnp.concatenate([src, jnp.zeros((pad,), _i32)])
    dst = jnp.concatenate([dst, jnp.full((pad,), N, _i32)])
    srcp, dstp, ecnt = _sc_part(src, dst)
    z = x
    ps2 = jnp.zeros((1, 1, D), _f32)
    pss2 = jnp.full((1, 1, D), N * (1.0 - 1e-5), _f32)
    gg = jnp.ones((1, D), _f32)
    bb = jnp.zeros((1, D), _f32)
    for l in range(L):
        q, k, v, xs = _qkvs(z, ps2, pss2, gg, bb,
                            Wq[l], bq[l].reshape(1, D),
                            Wk[l], bk[l].reshape(1, D),
                            Wv[l], bv[l].reshape(1, D),
                            Ws[l], bs[l].reshape(1, D))
        acc, den = _sc_edge(q, k, v, srcp, dstp, ecnt)
        y, ps, pss = _combine(acc, den, xs)
        z, ps2, pss2 = _ffn(y, ps, pss, bn1_g[l].reshape(1, D),
                            bn1_b[l].reshape(1, D), W1[l],
                            b1[l].reshape(1, F), W2[l], b2[l].reshape(1, D))
        gg = bn2_g[l].reshape(1, D)
        bb = bn2_b[l].reshape(1, D)
    return _apply(z, ps2, pss2, gg, bb)
